# baseline (speedup 1.0000x reference)
import functools

import jax
import jax.numpy as jnp
from jax import lax
from jax.experimental import pallas as pl
from jax.experimental.pallas import tpu as pltpu
from jax.experimental.pallas import tpu_sc as plsc

B, N, PD, K, OUT = 8, 2048, 2, 20, 10
CB = 8
BN = 256
CN = 512
NW = 32
IDXW = 128
NCHUNK = CB * N * K // IDXW


def _s2(a, b):
    return jnp.minimum(a, b), jnp.maximum(a, b)


def _s4(a, b):
    s1 = jnp.minimum(a[0], b[0])
    t = jnp.maximum(a[0], b[0])
    u = jnp.minimum(a[1], b[1])
    s4 = jnp.maximum(a[1], b[1])
    return s1, jnp.minimum(t, u), jnp.maximum(t, u), s4


def _m45(a, b):
    m1 = jnp.minimum(a[0], b[3])
    m2 = jnp.minimum(a[1], b[2])
    m3 = jnp.minimum(a[2], b[1])
    m4 = jnp.minimum(a[3], b[0])
    p5 = jnp.minimum(
        jnp.minimum(jnp.maximum(a[0], b[3]), jnp.maximum(a[1], b[2])),
        jnp.minimum(jnp.maximum(a[2], b[1]), jnp.maximum(a[3], b[0])))
    x1, x3 = _s2(m1, m3)
    x2, x4 = _s2(m2, m4)
    r1, r2 = _s2(x1, x2)
    r3, r4 = _s2(x3, x4)
    return r1, r2, r3, r4, p5


def _top5of8(cols):
    p01 = _s2(cols[0], cols[1])
    p23 = _s2(cols[2], cols[3])
    p45 = _s2(cols[4], cols[5])
    p67 = _s2(cols[6], cols[7])
    return _m45(_s4(p01, p23), _s4(p45, p67))


def _topk_write(distT, out_ref, base):
    key = lax.bitcast_convert_type(distT, jnp.int32)
    key = key ^ ((key >> 31) & jnp.int32(0x7FFFFFFF))
    row = lax.broadcasted_iota(jnp.int32, distT.shape, 0)
    key = (key & jnp.int32(~2047)) | row
    s = [key[c * 128:(c + 1) * 128, :] for c in range(16)]
    p1a, p2a, p3a, p4a, p5a = _top5of8(s[:8])
    p1b, p2b, p3b, p4b, p5b = _top5of8(s[8:])
    inf = jnp.int32(0x7FFFFFFF)
    tau = None
    for t in range(K):
        m = jnp.min(jnp.minimum(p1a, p1b), axis=0)
        out_ref[0, t, :] = (m & 2047) + base
        tau = m
        mb = m[None, :]
        sa = p1a == mb
        sb = p1b == mb
        p1a = jnp.where(sa, p2a, p1a)
        p2a = jnp.where(sa, p3a, p2a)
        p3a = jnp.where(sa, p4a, p3a)
        p4a = jnp.where(sa, inf, p4a)
        p1b = jnp.where(sb, p2b, p1b)
        p2b = jnp.where(sb, p3b, p2b)
        p3b = jnp.where(sb, p4b, p3b)
        p4b = jnp.where(sb, inf, p4b)
    tb = tau[None, :]
    bad = jnp.logical_or(jnp.any(p5a <= tb), jnp.any(p5b <= tb))

    @pl.when(bad)
    def _():
        kk = key
        for t in range(K):
            m = jnp.min(kk, axis=0)
            out_ref[0, t, :] = (m & 2047) + base
            kk = jnp.where(kk == m[None, :], inf, kk)


def _knn1_kernel(dP_ref, dT_ref, out_ref):
    b = pl.program_id(0)
    xc0 = dP_ref[0, :, 0:1]
    xc1 = dP_ref[0, :, 1:2]
    xr0 = dT_ref[0, 0:1, :]
    xr1 = dT_ref[0, 1:2, :]
    d2c = xc0 * xc0 + xc1 * xc1
    distT = d2c - 2.0 * (xc0 * xr0 + xc1 * xr1)
    _topk_write(distT, out_ref, b * N)


def _knn1(dT, dP, interpret=False):
    return pl.pallas_call(
        _knn1_kernel,
        grid=(CB, N // BN),
        in_specs=[
            pl.BlockSpec((1, N, 16), lambda b, n: (b, 0, 0)),
            pl.BlockSpec((1, 8, BN), lambda b, n: (b, 0, n)),
        ],
        out_specs=pl.BlockSpec((1, K, BN), lambda b, n: (b, 0, n)),
        out_shape=jax.ShapeDtypeStruct((CB, K, N), jnp.int32),
        interpret=interpret,
    )(dP, dT)


def _knn2_kernel(x1f_ref, x1b_ref, Wq_ref, Wp_ref, b2_ref,
                 idx_ref, q_ref, p_ref):
    b = pl.program_id(0)
    xf = x1f_ref[0]
    xr = x1b_ref[0]
    d2c = jnp.sum(xf * xf, axis=1, keepdims=True)
    g = lax.dot_general(xf.astype(jnp.bfloat16), xr.astype(jnp.bfloat16),
                        (((1,), (1,)), ((), ())),
                        preferred_element_type=jnp.float32)
    distT = d2c - 2.0 * g
    _topk_write(distT, idx_ref, b * N)
    q_ref[0] = jnp.dot(xr, Wq_ref[...], preferred_element_type=jnp.float32)
    p_ref[0] = (jnp.dot(xr, Wp_ref[...], preferred_element_type=jnp.float32)
                + b2_ref[...])


def _knn2(x1, Wq, Wp, b2, interpret=False):
    return pl.pallas_call(
        _knn2_kernel,
        grid=(CB, N // BN),
        in_specs=[
            pl.BlockSpec((1, N, 64), lambda b, n: (b, 0, 0)),
            pl.BlockSpec((1, BN, 64), lambda b, n: (b, n, 0)),
            pl.BlockSpec((64, 128), lambda b, n: (0, 0)),
            pl.BlockSpec((64, 128), lambda b, n: (0, 0)),
            pl.BlockSpec((1, 128), lambda b, n: (0, 0)),
        ],
        out_specs=[
            pl.BlockSpec((1, K, BN), lambda b, n: (b, 0, n)),
            pl.BlockSpec((1, BN, 128), lambda b, n: (b, n, 0)),
            pl.BlockSpec((1, BN, 128), lambda b, n: (b, n, 0)),
        ],
        out_shape=[
            jax.ShapeDtypeStruct((CB, K, N), jnp.int32),
            jax.ShapeDtypeStruct((CB, N, 128), jnp.float32),
            jax.ShapeDtypeStruct((CB, N, 128), jnp.float32),
        ],
        interpret=interpret,
    )(x1, x1, Wq, Wp, b2)


def _conv1_kernel(dP_ref, g_ref, W1_ref, b1_ref, W2_ref, b2_ref,
                  W3_ref, b3_ref, out_ref):
    xi0 = dP_ref[0, :, 0:1]
    xi1 = dP_ref[0, :, 1:2]
    xj0 = g_ref[0, :, :, 0:1]
    xj1 = g_ref[0, :, :, 1:2]
    w = W1_ref[...]
    w2 = w[2:3, :].reshape(1, 1, 64)
    w3 = w[3:4, :].reshape(1, 1, 64)
    zi = (xi0 * (w[0:1, :] - w[2:3, :]) + xi1 * (w[1:2, :] - w[3:4, :])
          + b1_ref[...])
    h1 = zi[None] + xj0 * w2 + xj1 * w3
    h1 = jnp.maximum(h1, 0.0).reshape(K * CN, 64)
    h2 = jnp.maximum(
        jnp.dot(h1, W2_ref[...], preferred_element_type=jnp.float32)
        + b2_ref[...], 0.0)
    h3 = jnp.maximum(
        jnp.dot(h2, W3_ref[...], preferred_element_type=jnp.float32)
        + b3_ref[...], 0.0)
    out_ref[0] = jnp.max(h3.reshape(K, CN, 64), axis=0)


def _conv1(dP, ptsg, W1, b1, W2, b2, W3, b3, interpret=False):
    return pl.pallas_call(
        _conv1_kernel,
        grid=(CB, N // CN),
        in_specs=[
            pl.BlockSpec((1, CN, 16), lambda b, n: (b, n, 0)),
            pl.BlockSpec((1, K, CN, 16), lambda b, n: (b, 0, n, 0)),
            pl.BlockSpec((4, 64), lambda b, n: (0, 0)),
            pl.BlockSpec((1, 64), lambda b, n: (0, 0)),
            pl.BlockSpec((64, 64), lambda b, n: (0, 0)),
            pl.BlockSpec((1, 64), lambda b, n: (0, 0)),
            pl.BlockSpec((64, 64), lambda b, n: (0, 0)),
            pl.BlockSpec((1, 64), lambda b, n: (0, 0)),
        ],
        out_specs=pl.BlockSpec((1, CN, 64), lambda b, n: (b, n, 0)),
        out_shape=jax.ShapeDtypeStruct((CB, N, 64), jnp.float32),
        interpret=interpret,
    )(dP, ptsg, W1, b1, W2, b2, W3, b3)


def _lin1pool_kernel(x1_ref, p_ref, qg_ref, Wa_ref, Wb_ref, bl_ref,
                     W1_ref, b1_ref, W2_ref, b2_ref, W3_ref, b3_ref,
                     pool_ref, out_ref):
    b = pl.program_id(0)
    nblk = pl.program_id(1)
    m = jnp.max(qg_ref[0], axis=0).astype(jnp.float32)
    x2 = jnp.maximum(p_ref[0] + m, 0.0)
    y = (jnp.dot(x1_ref[0], Wa_ref[...], preferred_element_type=jnp.float32)
         + jnp.dot(x2, Wb_ref[...], preferred_element_type=jnp.float32)
         + bl_ref[...])
    cm = jnp.max(y, axis=0, keepdims=True)

    @pl.when(nblk == 0)
    def _():
        pool_ref[pl.ds(b, 1), :] = cm

    @pl.when(nblk != 0)
    def _():
        pool_ref[pl.ds(b, 1), :] = jnp.maximum(pool_ref[pl.ds(b, 1), :], cm)

    @pl.when(jnp.logical_and(b == CB - 1, nblk == N // CN - 1))
    def _():
        h = jnp.maximum(
            jnp.dot(pool_ref[...], W1_ref[...],
                    preferred_element_type=jnp.float32) + b1_ref[...], 0.0)
        h = jnp.maximum(
            jnp.dot(h, W2_ref[...], preferred_element_type=jnp.float32)
            + b2_ref[...], 0.0)
        o = (jnp.dot(h, W3_ref[...], preferred_element_type=jnp.float32)
             + b3_ref[...])
        om = o - jnp.max(o, axis=1, keepdims=True)
        out_ref[...] = om - jnp.log(
            jnp.sum(jnp.exp(om), axis=1, keepdims=True))


def _lin1pool(x1, p, qg, Wa, Wb, bl, W1, b1, W2, b2, W3, b3,
              interpret=False):
    return pl.pallas_call(
        _lin1pool_kernel,
        grid=(CB, N // CN),
        in_specs=[
            pl.BlockSpec((1, CN, 64), lambda b, n: (b, n, 0)),
            pl.BlockSpec((1, CN, 128), lambda b, n: (b, n, 0)),
            pl.BlockSpec((1, K, CN, 128), lambda b, n: (b, 0, n, 0)),
            pl.BlockSpec((64, 1024), lambda b, n: (0, 0)),
            pl.BlockSpec((128, 1024), lambda b, n: (0, 0)),
            pl.BlockSpec((1, 1024), lambda b, n: (0, 0)),
            pl.BlockSpec((1024, 512), lambda b, n: (0, 0)),
            pl.BlockSpec((1, 512), lambda b, n: (0, 0)),
            pl.BlockSpec((512, 256), lambda b, n: (0, 0)),
            pl.BlockSpec((1, 256), lambda b, n: (0, 0)),
            pl.BlockSpec((256, OUT), lambda b, n: (0, 0)),
            pl.BlockSpec((1, OUT), lambda b, n: (0, 0)),
        ],
        out_specs=[
            pl.BlockSpec((CB, 1024), lambda b, n: (0, 0)),
            pl.BlockSpec((CB, OUT), lambda b, n: (0, 0)),
        ],
        out_shape=[
            jax.ShapeDtypeStruct((CB, 1024), jnp.float32),
            jax.ShapeDtypeStruct((CB, OUT), jnp.float32),
        ],
        interpret=interpret,
    )(x1, p, qg, Wa, Wb, bl, W1, b1, W2, b2, W3, b3)


@functools.lru_cache(maxsize=None)
def _sc_gather_build(D, dtype_name):
    dtype = jnp.dtype(dtype_name)
    cpw = NCHUNK // NW
    nbuf = 4
    mesh = plsc.VectorSubcoreMesh(core_axis_name="c", subcore_axis_name="s")

    @functools.partial(
        pl.kernel,
        out_type=jax.ShapeDtypeStruct((NCHUNK, IDXW, D), dtype),
        mesh=mesh,
        scratch_types=[
            pltpu.VMEM((cpw, IDXW), jnp.int32),
            pltpu.VMEM((nbuf, IDXW, D), dtype),
            pltpu.SemaphoreType.DMA((nbuf,)),
        ],
        compiler_params=pltpu.CompilerParams(use_tc_tiling_on_sc=False),
    )
    def gather_k(table_hbm, idx_hbm, out_hbm, idx_v, rows_v, sems):
        wid = lax.axis_index("s") * 2 + lax.axis_index("c")
        base = wid * cpw
        pltpu.sync_copy(idx_hbm.at[pl.ds(base, cpw)], idx_v)
        for bb in range(nbuf):
            pltpu.make_async_copy(
                table_hbm.at[idx_v.at[bb]], rows_v.at[bb], sems.at[bb]).start()

        def body(i, carry):
            for bb in range(nbuf):
                c = i * nbuf + bb
                pltpu.make_async_copy(
                    table_hbm.at[idx_v.at[c]], rows_v.at[bb],
                    sems.at[bb]).wait()
                pltpu.sync_copy(rows_v.at[bb], out_hbm.at[base + c])
                nxt = c + nbuf

                @pl.when(nxt < cpw)
                def _():
                    pltpu.make_async_copy(
                        table_hbm.at[idx_v.at[nxt]], rows_v.at[bb],
                        sems.at[bb]).start()
            return carry

        lax.fori_loop(0, cpw // nbuf, body, 0)

    return gather_k


def _gather_rows(table, idx2d):
    return _sc_gather_build(table.shape[-1], table.dtype.name)(table, idx2d)


def kernel(data, c1_W1, c1_b1, c1_W2, c1_b2, c1_W3, c1_b3, c2_W1, c2_b1,
           lin1_W, lin1_b, m_W1, m_b1, m_W2, m_b2, m_W3, m_b3):
    f32 = jnp.float32
    data = data.astype(f32)
    dP = jnp.pad(data, ((0, 0), (0, 0), (0, 16 - PD)))
    dT = jnp.pad(data.transpose(0, 2, 1), ((0, 0), (0, 8 - PD), (0, 0)))
    Wq = c2_W1[64:]
    Wp = c2_W1[:64] - c2_W1[64:]
    idx1 = _knn1(dT, dP)
    ptsg = _gather_rows(dP.reshape(CB * N, 16), idx1.reshape(NCHUNK, IDXW))
    ptsg = ptsg.reshape(CB, K, N, 16)
    x1 = _conv1(dP, ptsg, c1_W1, c1_b1.reshape(1, 64), c1_W2,
                c1_b2.reshape(1, 64), c1_W3, c1_b3.reshape(1, 64))
    idx2, q, p = _knn2(x1, Wq, Wp, c2_b1.reshape(1, 128))
    qg = _gather_rows(q.reshape(CB * N, 128), idx2.reshape(NCHUNK, IDXW))
    qg = qg.reshape(CB, K, N, 128)
    _, out = _lin1pool(x1, p, qg, lin1_W[:64], lin1_W[64:],
                       lin1_b.reshape(1, 1024), m_W1, m_b1.reshape(1, 512),
                       m_W2, m_b2.reshape(1, 256), m_W3,
                       m_b3.reshape(1, OUT))
    return out


# speedup vs baseline: 1.0047x; 1.0047x over previous
import functools

import jax
import jax.numpy as jnp
from jax import lax
from jax.experimental import pallas as pl
from jax.experimental.pallas import tpu as pltpu
from jax.experimental.pallas import tpu_sc as plsc

B, N, PD, K, OUT = 8, 2048, 2, 20, 10
CB = 8
BN = 512
CN = 512
NW = 32
IDXW = 128
NCHUNK = CB * N * K // IDXW


def _s2(a, b):
    return jnp.minimum(a, b), jnp.maximum(a, b)


def _s4(a, b):
    s1 = jnp.minimum(a[0], b[0])
    t = jnp.maximum(a[0], b[0])
    u = jnp.minimum(a[1], b[1])
    s4 = jnp.maximum(a[1], b[1])
    return s1, jnp.minimum(t, u), jnp.maximum(t, u), s4


def _m45(a, b):
    m1 = jnp.minimum(a[0], b[3])
    m2 = jnp.minimum(a[1], b[2])
    m3 = jnp.minimum(a[2], b[1])
    m4 = jnp.minimum(a[3], b[0])
    p5 = jnp.minimum(
        jnp.minimum(jnp.maximum(a[0], b[3]), jnp.maximum(a[1], b[2])),
        jnp.minimum(jnp.maximum(a[2], b[1]), jnp.maximum(a[3], b[0])))
    x1, x3 = _s2(m1, m3)
    x2, x4 = _s2(m2, m4)
    r1, r2 = _s2(x1, x2)
    r3, r4 = _s2(x3, x4)
    return r1, r2, r3, r4, p5


def _top5of8(cols):
    p01 = _s2(cols[0], cols[1])
    p23 = _s2(cols[2], cols[3])
    p45 = _s2(cols[4], cols[5])
    p67 = _s2(cols[6], cols[7])
    return _m45(_s4(p01, p23), _s4(p45, p67))


def _topk_write(distT, out_ref, base):
    key = lax.bitcast_convert_type(distT, jnp.int32)
    key = key ^ ((key >> 31) & jnp.int32(0x7FFFFFFF))
    row = lax.broadcasted_iota(jnp.int32, distT.shape, 0)
    key = (key & jnp.int32(~2047)) | row
    s = [key[c * 128:(c + 1) * 128, :] for c in range(16)]
    p1a, p2a, p3a, p4a, p5a = _top5of8(s[:8])
    p1b, p2b, p3b, p4b, p5b = _top5of8(s[8:])
    inf = jnp.int32(0x7FFFFFFF)
    tau = None
    for t in range(K):
        m = jnp.min(jnp.minimum(p1a, p1b), axis=0)
        out_ref[0, t, :] = (m & 2047) + base
        tau = m
        mb = m[None, :]
        sa = p1a == mb
        sb = p1b == mb
        p1a = jnp.where(sa, p2a, p1a)
        p2a = jnp.where(sa, p3a, p2a)
        p3a = jnp.where(sa, p4a, p3a)
        p4a = jnp.where(sa, inf, p4a)
        p1b = jnp.where(sb, p2b, p1b)
        p2b = jnp.where(sb, p3b, p2b)
        p3b = jnp.where(sb, p4b, p3b)
        p4b = jnp.where(sb, inf, p4b)
    tb = tau[None, :]
    bad = jnp.logical_or(jnp.any(p5a <= tb), jnp.any(p5b <= tb))

    @pl.when(bad)
    def _():
        kk = key
        for t in range(K):
            m = jnp.min(kk, axis=0)
            out_ref[0, t, :] = (m & 2047) + base
            kk = jnp.where(kk == m[None, :], inf, kk)


def _knn1_kernel(dP_ref, dT_ref, out_ref):
    b = pl.program_id(0)
    xc0 = dP_ref[0, :, 0:1]
    xc1 = dP_ref[0, :, 1:2]
    xr0 = dT_ref[0, 0:1, :]
    xr1 = dT_ref[0, 1:2, :]
    d2c = xc0 * xc0 + xc1 * xc1
    distT = d2c - 2.0 * (xc0 * xr0 + xc1 * xr1)
    _topk_write(distT, out_ref, b * N)


def _knn1(dT, dP, interpret=False):
    return pl.pallas_call(
        _knn1_kernel,
        grid=(CB, N // BN),
        in_specs=[
            pl.BlockSpec((1, N, 16), lambda b, n: (b, 0, 0)),
            pl.BlockSpec((1, 8, BN), lambda b, n: (b, 0, n)),
        ],
        out_specs=pl.BlockSpec((1, K, BN), lambda b, n: (b, 0, n)),
        out_shape=jax.ShapeDtypeStruct((CB, K, N), jnp.int32),
        interpret=interpret,
    )(dP, dT)


def _knn2_kernel(x1f_ref, x1b_ref, Wq_ref, Wp_ref, b2_ref,
                 idx_ref, q_ref, p_ref):
    b = pl.program_id(0)
    xf = x1f_ref[0]
    xr = x1b_ref[0]
    d2c = jnp.sum(xf * xf, axis=1, keepdims=True)
    g = lax.dot_general(xf.astype(jnp.bfloat16), xr.astype(jnp.bfloat16),
                        (((1,), (1,)), ((), ())),
                        preferred_element_type=jnp.float32)
    distT = d2c - 2.0 * g
    _topk_write(distT, idx_ref, b * N)
    q_ref[0] = jnp.dot(xr, Wq_ref[...], preferred_element_type=jnp.float32)
    p_ref[0] = (jnp.dot(xr, Wp_ref[...], preferred_element_type=jnp.float32)
                + b2_ref[...])


def _knn2(x1, Wq, Wp, b2, interpret=False):
    return pl.pallas_call(
        _knn2_kernel,
        grid=(CB, N // BN),
        in_specs=[
            pl.BlockSpec((1, N, 64), lambda b, n: (b, 0, 0)),
            pl.BlockSpec((1, BN, 64), lambda b, n: (b, n, 0)),
            pl.BlockSpec((64, 128), lambda b, n: (0, 0)),
            pl.BlockSpec((64, 128), lambda b, n: (0, 0)),
            pl.BlockSpec((1, 128), lambda b, n: (0, 0)),
        ],
        out_specs=[
            pl.BlockSpec((1, K, BN), lambda b, n: (b, 0, n)),
            pl.BlockSpec((1, BN, 128), lambda b, n: (b, n, 0)),
            pl.BlockSpec((1, BN, 128), lambda b, n: (b, n, 0)),
        ],
        out_shape=[
            jax.ShapeDtypeStruct((CB, K, N), jnp.int32),
            jax.ShapeDtypeStruct((CB, N, 128), jnp.float32),
            jax.ShapeDtypeStruct((CB, N, 128), jnp.float32),
        ],
        interpret=interpret,
    )(x1, x1, Wq, Wp, b2)


def _conv1_kernel(dP_ref, g_ref, W1_ref, b1_ref, W2_ref, b2_ref,
                  W3_ref, b3_ref, out_ref):
    xi0 = dP_ref[0, :, 0:1]
    xi1 = dP_ref[0, :, 1:2]
    xj0 = g_ref[0, :, :, 0:1]
    xj1 = g_ref[0, :, :, 1:2]
    w = W1_ref[...]
    w2 = w[2:3, :].reshape(1, 1, 64)
    w3 = w[3:4, :].reshape(1, 1, 64)
    zi = (xi0 * (w[0:1, :] - w[2:3, :]) + xi1 * (w[1:2, :] - w[3:4, :])
          + b1_ref[...])
    h1 = zi[None] + xj0 * w2 + xj1 * w3
    h1 = jnp.maximum(h1, 0.0).reshape(K * CN, 64)
    h2 = jnp.maximum(
        jnp.dot(h1, W2_ref[...], preferred_element_type=jnp.float32)
        + b2_ref[...], 0.0)
    h3 = jnp.maximum(
        jnp.dot(h2, W3_ref[...], preferred_element_type=jnp.float32)
        + b3_ref[...], 0.0)
    out_ref[0] = jnp.max(h3.reshape(K, CN, 64), axis=0)


def _conv1(dP, ptsg, W1, b1, W2, b2, W3, b3, interpret=False):
    return pl.pallas_call(
        _conv1_kernel,
        grid=(CB, N // CN),
        in_specs=[
            pl.BlockSpec((1, CN, 16), lambda b, n: (b, n, 0)),
            pl.BlockSpec((1, K, CN, 16), lambda b, n: (b, 0, n, 0)),
            pl.BlockSpec((4, 64), lambda b, n: (0, 0)),
            pl.BlockSpec((1, 64), lambda b, n: (0, 0)),
            pl.BlockSpec((64, 64), lambda b, n: (0, 0)),
            pl.BlockSpec((1, 64), lambda b, n: (0, 0)),
            pl.BlockSpec((64, 64), lambda b, n: (0, 0)),
            pl.BlockSpec((1, 64), lambda b, n: (0, 0)),
        ],
        out_specs=pl.BlockSpec((1, CN, 64), lambda b, n: (b, n, 0)),
        out_shape=jax.ShapeDtypeStruct((CB, N, 64), jnp.float32),
        interpret=interpret,
    )(dP, ptsg, W1, b1, W2, b2, W3, b3)


def _lin1pool_kernel(x1_ref, p_ref, qg_ref, Wa_ref, Wb_ref, bl_ref,
                     W1_ref, b1_ref, W2_ref, b2_ref, W3_ref, b3_ref,
                     pool_ref, out_ref):
    b = pl.program_id(0)
    nblk = pl.program_id(1)
    m = jnp.max(qg_ref[0], axis=0).astype(jnp.float32)
    x2 = jnp.maximum(p_ref[0] + m, 0.0)
    y = (jnp.dot(x1_ref[0], Wa_ref[...], preferred_element_type=jnp.float32)
         + jnp.dot(x2, Wb_ref[...], preferred_element_type=jnp.float32)
         + bl_ref[...])
    cm = jnp.max(y, axis=0, keepdims=True)

    @pl.when(nblk == 0)
    def _():
        pool_ref[pl.ds(b, 1), :] = cm

    @pl.when(nblk != 0)
    def _():
        pool_ref[pl.ds(b, 1), :] = jnp.maximum(pool_ref[pl.ds(b, 1), :], cm)

    @pl.when(jnp.logical_and(b == CB - 1, nblk == N // CN - 1))
    def _():
        h = jnp.maximum(
            jnp.dot(pool_ref[...], W1_ref[...],
                    preferred_element_type=jnp.float32) + b1_ref[...], 0.0)
        h = jnp.maximum(
            jnp.dot(h, W2_ref[...], preferred_element_type=jnp.float32)
            + b2_ref[...], 0.0)
        o = (jnp.dot(h, W3_ref[...], preferred_element_type=jnp.float32)
             + b3_ref[...])
        om = o - jnp.max(o, axis=1, keepdims=True)
        out_ref[...] = om - jnp.log(
            jnp.sum(jnp.exp(om), axis=1, keepdims=True))


def _lin1pool(x1, p, qg, Wa, Wb, bl, W1, b1, W2, b2, W3, b3,
              interpret=False):
    return pl.pallas_call(
        _lin1pool_kernel,
        grid=(CB, N // CN),
        in_specs=[
            pl.BlockSpec((1, CN, 64), lambda b, n: (b, n, 0)),
            pl.BlockSpec((1, CN, 128), lambda b, n: (b, n, 0)),
            pl.BlockSpec((1, K, CN, 128), lambda b, n: (b, 0, n, 0)),
            pl.BlockSpec((64, 1024), lambda b, n: (0, 0)),
            pl.BlockSpec((128, 1024), lambda b, n: (0, 0)),
            pl.BlockSpec((1, 1024), lambda b, n: (0, 0)),
            pl.BlockSpec((1024, 512), lambda b, n: (0, 0)),
            pl.BlockSpec((1, 512), lambda b, n: (0, 0)),
            pl.BlockSpec((512, 256), lambda b, n: (0, 0)),
            pl.BlockSpec((1, 256), lambda b, n: (0, 0)),
            pl.BlockSpec((256, OUT), lambda b, n: (0, 0)),
            pl.BlockSpec((1, OUT), lambda b, n: (0, 0)),
        ],
        out_specs=[
            pl.BlockSpec((CB, 1024), lambda b, n: (0, 0)),
            pl.BlockSpec((CB, OUT), lambda b, n: (0, 0)),
        ],
        out_shape=[
            jax.ShapeDtypeStruct((CB, 1024), jnp.float32),
            jax.ShapeDtypeStruct((CB, OUT), jnp.float32),
        ],
        interpret=interpret,
    )(x1, p, qg, Wa, Wb, bl, W1, b1, W2, b2, W3, b3)


@functools.lru_cache(maxsize=None)
def _sc_gather_build(D, dtype_name):
    dtype = jnp.dtype(dtype_name)
    cpw = NCHUNK // NW
    nbuf = 4
    mesh = plsc.VectorSubcoreMesh(core_axis_name="c", subcore_axis_name="s")

    @functools.partial(
        pl.kernel,
        out_type=jax.ShapeDtypeStruct((NCHUNK, IDXW, D), dtype),
        mesh=mesh,
        scratch_types=[
            pltpu.VMEM((cpw, IDXW), jnp.int32),
            pltpu.VMEM((nbuf, IDXW, D), dtype),
            pltpu.SemaphoreType.DMA((nbuf,)),
        ],
        compiler_params=pltpu.CompilerParams(use_tc_tiling_on_sc=False),
    )
    def gather_k(table_hbm, idx_hbm, out_hbm, idx_v, rows_v, sems):
        wid = lax.axis_index("s") * 2 + lax.axis_index("c")
        base = wid * cpw
        pltpu.sync_copy(idx_hbm.at[pl.ds(base, cpw)], idx_v)
        for bb in range(nbuf):
            pltpu.make_async_copy(
                table_hbm.at[idx_v.at[bb]], rows_v.at[bb], sems.at[bb]).start()

        def body(i, carry):
            for bb in range(nbuf):
                c = i * nbuf + bb
                pltpu.make_async_copy(
                    table_hbm.at[idx_v.at[c]], rows_v.at[bb],
                    sems.at[bb]).wait()
                pltpu.sync_copy(rows_v.at[bb], out_hbm.at[base + c])
                nxt = c + nbuf

                @pl.when(nxt < cpw)
                def _():
                    pltpu.make_async_copy(
                        table_hbm.at[idx_v.at[nxt]], rows_v.at[bb],
                        sems.at[bb]).start()
            return carry

        lax.fori_loop(0, cpw // nbuf, body, 0)

    return gather_k


def _gather_rows(table, idx2d):
    return _sc_gather_build(table.shape[-1], table.dtype.name)(table, idx2d)


def kernel(data, c1_W1, c1_b1, c1_W2, c1_b2, c1_W3, c1_b3, c2_W1, c2_b1,
           lin1_W, lin1_b, m_W1, m_b1, m_W2, m_b2, m_W3, m_b3):
    f32 = jnp.float32
    data = data.astype(f32)
    dP = jnp.pad(data, ((0, 0), (0, 0), (0, 16 - PD)))
    dT = jnp.pad(data.transpose(0, 2, 1), ((0, 0), (0, 8 - PD), (0, 0)))
    Wq = c2_W1[64:]
    Wp = c2_W1[:64] - c2_W1[64:]
    idx1 = _knn1(dT, dP)
    ptsg = _gather_rows(dP.reshape(CB * N, 16), idx1.reshape(NCHUNK, IDXW))
    ptsg = ptsg.reshape(CB, K, N, 16)
    x1 = _conv1(dP, ptsg, c1_W1, c1_b1.reshape(1, 64), c1_W2,
                c1_b2.reshape(1, 64), c1_W3, c1_b3.reshape(1, 64))
    idx2, q, p = _knn2(x1, Wq, Wp, c2_b1.reshape(1, 128))
    qg = _gather_rows(q.reshape(CB * N, 128), idx2.reshape(NCHUNK, IDXW))
    qg = qg.reshape(CB, K, N, 128)
    _, out = _lin1pool(x1, p, qg, lin1_W[:64], lin1_W[64:],
                       lin1_b.reshape(1, 1024), m_W1, m_b1.reshape(1, 512),
                       m_W2, m_b2.reshape(1, 256), m_W3,
                       m_b3.reshape(1, OUT))
    return out
